# Initial kernel scaffold; baseline (speedup 1.0000x reference)
import jax
import jax.numpy as jnp
from jax.experimental import pallas as pl


def kernel(x, edge_index, edge_attr, masks, W1, b1, W2, b2, eps, gamma, beta):
    raise NotImplementedError("write your pallas kernel here")


# speedup vs baseline: 3.2303x; 3.2303x over previous
import functools

import jax
import jax.numpy as jnp
from jax import lax
from jax.experimental import pallas as pl
from jax.experimental.pallas import tpu as pltpu
from jax.experimental.pallas import tpu_sc as plsc

NC = 2
NS = 16
NW = NC * NS
CH = 80


def _make_sc_agg(n, e, d, nch):
    ts = (n // NS) // 8 * 8
    rem = n - ts * NS
    assert rem % 8 == 0
    mesh = plsc.VectorSubcoreMesh(core_axis_name="c", subcore_axis_name="s")

    @functools.partial(
        pl.kernel,
        out_type=jax.ShapeDtypeStruct((NC, n, d), jnp.float32),
        mesh=mesh,
        scratch_types=[
            pltpu.VMEM((1, CH), jnp.int32),
            pltpu.VMEM((1, CH), jnp.int32),
            pltpu.VMEM((CH, d), jnp.float32),
            pltpu.VMEM((CH, d), jnp.float32),
            pltpu.VMEM_SHARED((n, d), jnp.float32),
            pltpu.SemaphoreType.DMA,
            pltpu.SemaphoreType.DMA,
        ],
    )
    def sc_agg(x_hbm, src_hbm, dst_hbm, ea_hbm, zero_hbm, out_hbm,
               src_v, dst_v, rows_v, ea_v, agg_sh, sem_g, sem_e):
        c = lax.axis_index("c")
        s = lax.axis_index("s")
        wid = c * NS + s

        pltpu.sync_copy(zero_hbm.at[pl.ds(s * ts, ts)],
                        agg_sh.at[pl.ds(s * ts, ts)])
        if rem:
            @pl.when(s == 0)
            def _():
                pltpu.sync_copy(zero_hbm.at[pl.ds(ts * NS, rem)],
                                agg_sh.at[pl.ds(ts * NS, rem)])
        plsc.subcore_barrier()

        def chunk(j, carry):
            pltpu.sync_copy(src_hbm.at[wid, j], src_v.at[0])
            pltpu.sync_copy(dst_hbm.at[wid, j], dst_v.at[0])
            g = pltpu.async_copy(x_hbm.at[src_v.at[0]], rows_v, sem_g)
            a = pltpu.async_copy(ea_hbm.at[wid, j], ea_v, sem_e)
            g.wait()
            a.wait()

            def row(r, carry2):
                for cc in range(d // 16):
                    sl = pl.ds(cc * 16, 16)
                    rows_v[r, sl] = jnp.maximum(
                        rows_v[r, sl] + ea_v[r, sl], 0.0)
                return carry2

            lax.fori_loop(0, CH, row, 0)
            pltpu.sync_copy(rows_v, agg_sh.at[dst_v.at[0]], add=True)
            return carry

        lax.fori_loop(0, nch, chunk, 0)
        plsc.subcore_barrier()
        pltpu.sync_copy(agg_sh.at[pl.ds(s * ts, ts)],
                        out_hbm.at[c, pl.ds(s * ts, ts)])
        if rem:
            @pl.when(s == 0)
            def _():
                pltpu.sync_copy(agg_sh.at[pl.ds(ts * NS, rem)],
                                out_hbm.at[c, pl.ds(ts * NS, rem)])

    return sc_agg


def _passA_body(eps_ref, x_ref, a0_ref, a1_ref, m_ref, w1_ref, b1_ref,
                w2_ref, b2_ref, out_ref, st_ref):
    x = x_ref[...]
    h0 = (1.0 + eps_ref[0, 0]) * x + (a0_ref[...] + a1_ref[...])
    h1 = jnp.maximum(
        jnp.dot(h0, w1_ref[...], preferred_element_type=jnp.float32,
                precision=lax.Precision.HIGHEST) + b1_ref[...], 0.0)
    h2 = jnp.dot(h1, w2_ref[...], preferred_element_type=jnp.float32,
                 precision=lax.Precision.HIGHEST) + b2_ref[...]
    xn = m_ref[...] * h2 + x
    out_ref[...] = xn
    ps = jnp.sum(xn, axis=0, keepdims=True)
    pq = jnp.sum(xn * xn, axis=0, keepdims=True)
    st = jnp.concatenate([ps, pq], axis=0)

    @pl.when(pl.program_id(0) == 0)
    def _():
        st_ref[...] = st

    @pl.when(pl.program_id(0) > 0)
    def _():
        st_ref[...] += st


def _passB_body(n, st_ref, g_ref, be_ref, xn_ref, out_ref):
    mean = st_ref[0:1, :] * (1.0 / n)
    msq = st_ref[1:2, :] * (1.0 / n)
    var = msq - mean * mean
    inv = lax.rsqrt(var + 1e-5)
    out_ref[...] = jnp.maximum(
        (xn_ref[...] - mean) * inv * g_ref[...] + be_ref[...], 0.0)


def _make_tc(n, d, bn):
    nb = n // bn
    grid = (nb,)
    row_spec = pl.BlockSpec((bn, d), lambda i: (i, 0))
    full_spec = pl.BlockSpec((d, d), lambda i: (0, 0))
    vec_spec = pl.BlockSpec((1, d), lambda i: (0, 0))

    passA = pl.pallas_call(
        _passA_body,
        grid=grid,
        in_specs=[
            pl.BlockSpec((1, 1), lambda i: (0, 0)),
            row_spec, row_spec, row_spec, row_spec,
            full_spec, vec_spec, full_spec, vec_spec,
        ],
        out_specs=[row_spec, pl.BlockSpec((2, d), lambda i: (0, 0))],
        out_shape=[
            jax.ShapeDtypeStruct((n, d), jnp.float32),
            jax.ShapeDtypeStruct((2, d), jnp.float32),
        ],
    )

    passB = pl.pallas_call(
        functools.partial(_passB_body, n),
        grid=grid,
        in_specs=[
            pl.BlockSpec((2, d), lambda i: (0, 0)),
            vec_spec, vec_spec,
            row_spec,
        ],
        out_specs=row_spec,
        out_shape=jax.ShapeDtypeStruct((n, d), jnp.float32),
    )
    return passA, passB


def kernel(x, edge_index, edge_attr, masks, W1, b1, W2, b2, eps, gamma, beta):
    n, d = x.shape
    e = edge_index.shape[1]
    nlayers = W1.shape[0]
    assert e % (NW * CH) == 0 and n % NS == 0 and d % 16 == 0
    nch = e // (NW * CH)

    src = edge_index[0].reshape(NW, nch, CH)
    dst = edge_index[1].reshape(NW, nch, CH)
    ear = edge_attr.reshape(NW, nch, CH, d)
    zero = jnp.zeros((n, d), jnp.float32)

    sc_agg = _make_sc_agg(n, e, d, nch)
    passA, passB = _make_tc(n, d, 400)

    for l in range(nlayers):
        aggp = sc_agg(x, src, dst, ear, zero)
        xn, st = passA(eps[l].reshape(1, 1), x, aggp[0], aggp[1],
                       jnp.broadcast_to(masks[l][:, None], (n, d)),
                       W1[l], b1[l].reshape(1, d), W2[l], b2[l].reshape(1, d))
        x = passB(st, gamma[l].reshape(1, d), beta[l].reshape(1, d), xn)
    return x
